# Initial kernel scaffold; baseline (speedup 1.0000x reference)
import jax
import jax.numpy as jnp
from jax.experimental import pallas as pl


def kernel(x, A0, B0, A1, B1, A2, B2, A3, B3, A4, B4, A5, B5, A6, B6, A7, B7):
    raise NotImplementedError("write your pallas kernel here")


# speedup vs baseline: 6.8730x; 6.8730x over previous
import jax
import jax.numpy as jnp
from jax.experimental import pallas as pl

_RANKS = (8, 16, 32, 8, 16, 32, 8, 16)
_ALPHA = 1.0
_RMAX = 32
_NUM_ADAPTERS = 8
_SBLK = 512


def _lora_kernel(x_ref, bt_ref, at_ref, o_ref):
    xb = x_ref[0]
    y = jnp.dot(xb, bt_ref[0], preferred_element_type=jnp.float32)
    o_ref[0] = jnp.dot(y, at_ref[0], preferred_element_type=jnp.float32)


def kernel(x, A0, B0, A1, B1, A2, B2, A3, B3, A4, B4, A5, B5, A6, B6, A7, B7):
    As = (A0, A1, A2, A3, A4, A5, A6, A7)
    Bs = (B0, B1, B2, B3, B4, B5, B6, B7)
    B, S, D = x.shape
    out_f = As[0].shape[0]

    bt = jnp.stack([
        jnp.pad(Bs[a].T, ((0, 0), (0, _RMAX - _RANKS[a]))) for a in range(_NUM_ADAPTERS)
    ])
    at = jnp.stack([
        jnp.pad((As[a] * (_ALPHA / _RANKS[a])).T, ((0, _RMAX - _RANKS[a]), (0, 0)))
        for a in range(_NUM_ADAPTERS)
    ])

    return pl.pallas_call(
        _lora_kernel,
        grid=(B, S // _SBLK),
        in_specs=[
            pl.BlockSpec((1, _SBLK, D), lambda b, s: (b, s, 0)),
            pl.BlockSpec((1, D, _RMAX), lambda b, s: (b % _NUM_ADAPTERS, 0, 0)),
            pl.BlockSpec((1, _RMAX, out_f), lambda b, s: (b % _NUM_ADAPTERS, 0, 0)),
        ],
        out_specs=pl.BlockSpec((1, _SBLK, D), lambda b, s: (b, s, 0)),
        out_shape=jax.ShapeDtypeStruct((B, S, out_f), x.dtype),
    )(x, bt, at)
